# baseline (speedup 1.0000x reference)
import functools
import jax
import jax.numpy as jnp
from jax import lax
from jax.experimental import pallas as pl
from jax.experimental.pallas import tpu as pltpu
from jax.experimental.pallas import tpu_sc as plsc


def kernel(pixel_values, row_weight, col_weight):
    if pixel_values.ndim != 4:
        raise ValueError('pixel_values must be a 4D tensor')
    b, h, w, _ = pixel_values.shape
    dr = row_weight.shape[1]
    dc = col_weight.shape[1]
    d = dc + dr

    row_w = row_weight[:h]
    col_w = col_weight[:w]

    nc, ns = 2, 16
    nw = nc * ns
    assert h == nw

    mesh = plsc.VectorSubcoreMesh(
        core_axis_name="c", subcore_axis_name="s", num_cores=nc, num_subcores=ns
    )

    @functools.partial(
        pl.kernel,
        mesh=mesh,
        out_type=jax.ShapeDtypeStruct((b, h, w, d), jnp.float32),
        scratch_types=[
            pltpu.VMEM((w, dc), jnp.float32),
            pltpu.VMEM((1, dr), jnp.float32),
            pltpu.VMEM((w, d), jnp.float32),
            pltpu.SemaphoreType.DMA,
        ],
    )
    def sc_k(col_hbm, row_hbm, out_hbm, col_v, row_v, plane_v, sem):
        wid = lax.axis_index("s") * nc + lax.axis_index("c")
        pltpu.sync_copy(col_hbm, col_v)
        pltpu.sync_copy(row_hbm.at[pl.ds(wid, 1)], row_v)

        def build_row(iw, carry):
            for j in range(dc // 16):
                plane_v[iw, pl.ds(j * 16, 16)] = col_v[iw, pl.ds(j * 16, 16)]
            for j in range(dr // 16):
                plane_v[iw, pl.ds(dc + j * 16, 16)] = row_v[0, pl.ds(j * 16, 16)]
            return carry

        lax.fori_loop(0, w, build_row, 0)

        copies = [
            pltpu.async_copy(plane_v, out_hbm.at[ib, wid], sem) for ib in range(b)
        ]
        for c in copies:
            c.wait()

    return sc_k(col_w, row_w)
